# Initial kernel scaffold; baseline (speedup 1.0000x reference)
import jax
import jax.numpy as jnp
from jax.experimental import pallas as pl


def kernel(edge_index, edge_weight, pol_features, state_ids, sector_ids, industry_ids, comp_scalar, W_pol, b_pol, state_emb, sector_emb, industry_emb, W_comp, b_comp, comm_emb, ln_g, ln_b, W1_rel, b1_rel, W1_root, W2_rel, b2_rel, W2_root):
    raise NotImplementedError("write your pallas kernel here")


# speedup vs baseline: 4.3042x; 4.3042x over previous
import functools

import jax
import jax.numpy as jnp
from jax import lax
from jax.experimental import pallas as pl
from jax.experimental.pallas import tpu as pltpu
from jax.experimental.pallas import tpu_sc as plsc

N_POL = 60000
N_TICK = 39000
N_COMM = 1000
N_NODES = N_POL + N_TICK + N_COMM
F = 32
E = 1600000

HALF = N_NODES // 2
N_TILES = 16
BLK = 512
ROWS_PER_BLK = BLK // 128
EPT_BLOCKS = 200
E_PAD = N_TILES * EPT_BLOCKS * BLK
ACC_ROWS = 50176
ZCH = 64
ZPT = ACC_ROWS // N_TILES
OPT = 3128


def _seg_body(x_hbm, src_hbm, dst_hbm, w_hbm, agg_hbm,
              acc, src_i, dst_i, w_flat, idx_v, rows, zbuf, sem):
    c = lax.axis_index("c")
    s = lax.axis_index("s")
    lo = c * HALF
    dump = HALF + s * 8

    zeros16 = jnp.zeros((16,), jnp.float32)

    @pl.loop(0, ZCH)
    def _zb(i):
        zbuf[i, pl.ds(0, 16)] = zeros16
        zbuf[i, pl.ds(16, 16)] = zeros16

    @pl.loop(0, ZPT // ZCH)
    def _zc(i):
        pltpu.sync_copy(zbuf, acc.at[pl.ds(s * ZPT + i * ZCH, ZCH)])

    plsc.subcore_barrier()

    @pl.loop(0, EPT_BLOCKS)
    def _block(b):
        row0 = s * (EPT_BLOCKS * ROWS_PER_BLK) + b * ROWS_PER_BLK
        ebase = (s * EPT_BLOCKS + b) * BLK
        pltpu.sync_copy(src_hbm.at[pl.ds(row0, ROWS_PER_BLK)], src_i)
        pltpu.sync_copy(dst_hbm.at[pl.ds(row0, ROWS_PER_BLK)], dst_i)
        pltpu.sync_copy(w_hbm.at[pl.ds(ebase, BLK)], w_flat.at[pl.ds(0, BLK)])

        cps = [pltpu.async_copy(x_hbm.at[src_i.at[j]],
                                rows.at[pl.ds(j * 128, 128)], sem)
               for j in range(ROWS_PER_BLK)]
        for cp in cps:
            cp.wait()

        @plsc.parallel_loop(0, ROWS_PER_BLK * 8, step=1)
        def _ib(g):
            j = g // 8
            k = (g % 8) * 16
            t = dst_i[j, pl.ds(k, 16)] - lo
            inb = (t >= 0) & (t < HALF)
            idx_v[j, pl.ds(k, 16)] = jnp.where(inb, t, dump)

        @plsc.parallel_loop(0, BLK, step=1)
        def _mb(e):
            wv = w_flat[pl.ds(e, 16)][0]
            rows[e, pl.ds(0, 16)] = rows[e, pl.ds(0, 16)] * wv
            rows[e, pl.ds(16, 16)] = rows[e, pl.ds(16, 16)] * wv

        for j in range(ROWS_PER_BLK):
            pltpu.sync_copy(rows.at[pl.ds(j * 128, 128)],
                            acc.at[idx_v.at[j]], add=True)

    plsc.subcore_barrier()
    base = s * OPT
    last = HALF - 15 * OPT

    @pl.when(s < 15)
    def _out_main():
        pltpu.sync_copy(acc.at[pl.ds(base, OPT)],
                        agg_hbm.at[pl.ds(lo + base, OPT)])

    @pl.when(s == 15)
    def _out_last():
        pltpu.sync_copy(acc.at[pl.ds(15 * OPT, last)],
                        agg_hbm.at[pl.ds(lo + 15 * OPT, last)])


_seg_kernel = pl.kernel(
    _seg_body,
    out_type=jax.ShapeDtypeStruct((N_NODES, F), jnp.float32),
    mesh=plsc.VectorSubcoreMesh(core_axis_name="c", subcore_axis_name="s"),
    compiler_params=pltpu.CompilerParams(use_tc_tiling_on_sc=False),
    scratch_types=[
        pltpu.VMEM_SHARED((ACC_ROWS, F), jnp.float32),
        pltpu.VMEM((ROWS_PER_BLK, 128), jnp.int32),
        pltpu.VMEM((ROWS_PER_BLK, 128), jnp.int32),
        pltpu.VMEM((BLK + 16,), jnp.float32),
        pltpu.VMEM((ROWS_PER_BLK, 128), jnp.int32),
        pltpu.VMEM((BLK, F), jnp.float32),
        pltpu.VMEM((ZCH, F), jnp.float32),
        pltpu.SemaphoreType.DMA,
    ],
)


def _segment_sum(x, src2d, dst2d, w2d):
    return _seg_kernel(x, src2d, dst2d, w2d)


def _ln(x, g, b):
    mu = jnp.mean(x, axis=-1, keepdims=True)
    d = x - mu
    var = jnp.mean(d * d, axis=-1, keepdims=True)
    return d / jnp.sqrt(var + 1e-5) * g + b


def _pol_body(pf_ref, ids_ref, wp_ref, bp_ref, semb_ref, g_ref, bln_ref, o_ref):
    h = jnp.dot(pf_ref[...], wp_ref[...], preferred_element_type=jnp.float32)
    h = jnp.maximum(h + bp_ref[...], 0.0)
    br = ids_ref.shape[0]
    oh = (lax.broadcasted_iota(jnp.int32, (br, 50), 1) == ids_ref[...]
          ).astype(jnp.float32)
    h = h + jnp.dot(oh, semb_ref[...], preferred_element_type=jnp.float32)
    o_ref[...] = _ln(h, g_ref[...], bln_ref[...])


def _comp_body(sid_ref, iid_ref, cs_ref, se_ref, ie_ref, ws_ref, wi_ref,
               wc_ref, bc_ref, g_ref, bln_ref, o_ref):
    br = sid_ref.shape[0]
    ohs = (lax.broadcasted_iota(jnp.int32, (br, 12), 1) == sid_ref[...]
           ).astype(jnp.float32)
    se = jnp.dot(ohs, se_ref[...], preferred_element_type=jnp.float32)
    ohi = (lax.broadcasted_iota(jnp.int32, (br, 152), 1) == iid_ref[...]
           ).astype(jnp.float32)
    ie = jnp.dot(ohi, ie_ref[...], preferred_element_type=jnp.float32)
    h = (jnp.dot(se, ws_ref[...], preferred_element_type=jnp.float32)
         + jnp.dot(ie, wi_ref[...], preferred_element_type=jnp.float32)
         + cs_ref[...] * wc_ref[...] + bc_ref[...])
    h = jnp.maximum(h, 0.0)
    o_ref[...] = _ln(h, g_ref[...], bln_ref[...])


def _comm_body(ce_ref, g_ref, bln_ref, o_ref):
    o_ref[...] = _ln(ce_ref[...], g_ref[...], bln_ref[...])


def _dense_body(relu, agg_ref, xin_ref, wr_ref, b_ref, wroot_ref, o_ref):
    acc = jnp.dot(agg_ref[...], wr_ref[...], preferred_element_type=jnp.float32)
    acc = acc + jnp.dot(xin_ref[...], wroot_ref[...],
                        preferred_element_type=jnp.float32)
    acc = acc + b_ref[...]
    if relu:
        acc = jnp.maximum(acc, 0.0)
    o_ref[...] = acc


def _row_spec(br, w):
    return pl.BlockSpec((br, w), lambda i: (i, 0))


def _full_spec(a, b):
    return pl.BlockSpec((a, b), lambda i: (0, 0))


def _dense(agg, xin, wr, b, wroot, relu):
    br = 4000
    return pl.pallas_call(
        functools.partial(_dense_body, relu),
        grid=(N_NODES // br,),
        in_specs=[_row_spec(br, F), _row_spec(br, F), _full_spec(F, F),
                  _full_spec(1, F), _full_spec(F, F)],
        out_specs=_row_spec(br, F),
        out_shape=jax.ShapeDtypeStruct((N_NODES, F), jnp.float32),
    )(agg, xin, wr, b.reshape(1, F), wroot)


def kernel(edge_index, edge_weight, pol_features, state_ids, sector_ids,
           industry_ids, comp_scalar, W_pol, b_pol, state_emb, sector_emb,
           industry_emb, W_comp, b_comp, comm_emb, ln_g, ln_b,
           W1_rel, b1_rel, W1_root, W2_rel, b2_rel, W2_root):
    g2 = ln_g.reshape(1, F)
    bln2 = ln_b.reshape(1, F)

    pf = jnp.pad(pol_features, ((0, 0), (0, 1)))
    wp = jnp.pad(W_pol, ((0, 1), (0, 0)))
    br_p = 3000
    xp = pl.pallas_call(
        _pol_body,
        grid=(N_POL // br_p,),
        in_specs=[_row_spec(br_p, 8), _row_spec(br_p, 1), _full_spec(8, F),
                  _full_spec(1, F), _full_spec(50, F), _full_spec(1, F),
                  _full_spec(1, F)],
        out_specs=_row_spec(br_p, F),
        out_shape=jax.ShapeDtypeStruct((N_POL, F), jnp.float32),
    )(pf, state_ids.reshape(N_POL, 1), wp, b_pol.reshape(1, F), state_emb,
      g2, bln2)

    br_c = 3000
    iemb = jnp.pad(industry_emb, ((0, 2), (0, 0)))
    xc = pl.pallas_call(
        _comp_body,
        grid=(N_TICK // br_c,),
        in_specs=[_row_spec(br_c, 1), _row_spec(br_c, 1), _row_spec(br_c, 1),
                  _full_spec(12, 8), _full_spec(152, 8), _full_spec(8, F),
                  _full_spec(8, F), _full_spec(1, F), _full_spec(1, F),
                  _full_spec(1, F), _full_spec(1, F)],
        out_specs=_row_spec(br_c, F),
        out_shape=jax.ShapeDtypeStruct((N_TICK, F), jnp.float32),
    )(sector_ids.reshape(N_TICK, 1), industry_ids.reshape(N_TICK, 1),
      comp_scalar, sector_emb, iemb, W_comp[0:8], W_comp[8:16],
      W_comp[16:17], b_comp.reshape(1, F), g2, bln2)

    xm = pl.pallas_call(
        _comm_body,
        grid=(1,),
        in_specs=[_row_spec(N_COMM, F), _full_spec(1, F), _full_spec(1, F)],
        out_specs=_row_spec(N_COMM, F),
        out_shape=jax.ShapeDtypeStruct((N_COMM, F), jnp.float32),
    )(comm_emb, g2, bln2)

    x = jnp.concatenate([xp, xc, xm], axis=0)

    src = edge_index[0]
    dst = edge_index[1]
    pad = E_PAD - E
    src2d = jnp.pad(src, (0, pad)).reshape(E_PAD // 128, 128)
    dst2d = jnp.pad(dst, (0, pad), constant_values=N_NODES
                    ).reshape(E_PAD // 128, 128)
    w2d = jnp.pad(edge_weight, (0, pad))

    agg1 = _segment_sum(x, src2d, dst2d, w2d)
    h = _dense(agg1, x, W1_rel, b1_rel, W1_root, True)

    agg2 = _segment_sum(h, src2d, dst2d, w2d)
    out = _dense(agg2, h, W2_rel, b2_rel, W2_root, False)
    return out
